# baseline (speedup 1.0000x reference)
import math

import jax
import jax.numpy as jnp
import numpy as np
from jax.experimental import pallas as pl
from jax.experimental.pallas import tpu as pltpu

_SC = sum(math.sin(c + 1) for c in range(5))
_SD = sum(1.0 / (math.cos(d + 1e-9) + 1e-9) for d in range(5))
_SE = sum(math.sqrt(e + 1) for e in range(5))
_SMUL = _SC * _SD * _SE

_CI = np.arange(23)
_CJ = (_CI * 7919) % 23
_CK = (_CJ * 1543) % 23

_POOL = 28 * 28 + 1e-9
_BBLK = 32

_XI = np.arange(32)
_DXM = _XI[:, None] - _XI[None, :] + 2
_DX_OK = (_DXM >= 0) & (_DXM < 5) & (_XI[None, :] < 28) & (_XI[:, None] < 28)
_DXC = np.clip(_DXM, 0, 4)


def _prep_body(sc_ref, tv_ref, b1_ref, n1_ref, b2_ref, n2_ref, w1_ref, w2_ref):
    def mk(base, noise, ab, ts):
        acc = base * _SMUL
        acc = acc + ab * jnp.sum(acc, axis=1, keepdims=True)
        acc = acc + ts
        r = noise
        for _ in range(3):
            r = r * (r + 1e-7)
        k = acc + r
        return k / (jnp.mean(jnp.abs(k)) + 1e-12)

    ts1 = jnp.sum(tv_ref[0:1, :]) * 1e-12
    ts2 = jnp.sum(tv_ref[1:2, :]) * 1e-12
    w1_ref[...] = mk(b1_ref[...], n1_ref[...], sc_ref[0], ts1)
    w2_ref[...] = mk(b2_ref[...], n2_ref[...], sc_ref[1], ts2)


def _wmix_body(fc_ref, lin_ref, out_ref):
    @pl.when(pl.program_id(0) == 0)
    def _init():
        out_ref[...] = jnp.zeros_like(out_ref)

    out_ref[...] += jnp.dot(
        fc_ref[...], lin_ref[...], preferred_element_type=jnp.float32
    )


def _rollrows(x, a):
    if a == 0:
        return x
    return jnp.roll(x, -a, axis=1)


def _conv_block(xb, g1_ref, g2_ref):
    h1 = jnp.einsum(
        "byk,kn->byn", xb, g1_ref[2], preferred_element_type=jnp.float32
    )
    for ai, a in ((0, -2), (1, -1), (3, 1), (4, 2)):
        h1 = h1 + jnp.einsum(
            "byk,kn->byn", _rollrows(xb, a), g1_ref[ai],
            preferred_element_type=jnp.float32,
        )
    yid = jax.lax.broadcasted_iota(jnp.int32, (1, 32, 1), 1)
    yok = (yid >= 2) & (yid < 30)
    x1 = jax.lax.broadcasted_iota(jnp.int32, (1, 1, 256), 2) % 32 < 28
    h1 = jnp.where(yok & x1, jnp.maximum(h1, 0.0) / _POOL, 0.0)

    h2 = jnp.einsum(
        "byk,kn->byn", h1, g2_ref[2], preferred_element_type=jnp.float32
    )
    for ai, a in ((0, -2), (1, -1), (3, 1), (4, 2)):
        h2 = h2 + jnp.einsum(
            "byk,kn->byn", _rollrows(h1, a), g2_ref[ai],
            preferred_element_type=jnp.float32,
        )
    x2 = jax.lax.broadcasted_iota(jnp.int32, (1, 1, 512), 2) % 32 < 28
    h2 = jnp.where(yok & x2, jnp.maximum(h2, 0.0) / _POOL, 0.0)

    ss = jnp.sum(h2 * h2, axis=(1, 2), keepdims=True)
    vn = h2 / (jnp.sqrt(ss) + 1e-20)
    return vn * (vn + 1e-12)


def _conv_body(x_ref, g1_ref, g2_ref, v_ref):
    v_ref[...] = _conv_block(x_ref[...], g1_ref, g2_ref)


def _fc_body(v_ref, w_ref, b_ref, y_ref):
    y = jnp.dot(
        v_ref[...], w_ref[...], preferred_element_type=jnp.float32
    ) + b_ref[...]
    m = jnp.mean(jnp.abs(y[:, :10]))
    y_ref[...] = y / (m + 1e-30)


def kernel(x, base1, a1, b1, thrash1, noise1, base2, a2, b2, thrash2, noise2,
           lin_w, fc_w, fc_b):
    f32 = jnp.float32

    sc = jnp.stack([a1 * b1, a2 * b2])
    tv = jnp.stack([thrash1[_CI, _CJ, _CK], thrash2[_CI, _CJ, _CK]])
    w1n, w2n = pl.pallas_call(
        _prep_body,
        in_specs=[
            pl.BlockSpec(memory_space=pltpu.SMEM),
            pl.BlockSpec(),
            pl.BlockSpec(),
            pl.BlockSpec(),
            pl.BlockSpec(),
            pl.BlockSpec(),
        ],
        out_shape=[
            jax.ShapeDtypeStruct((8, 25), f32),
            jax.ShapeDtypeStruct((128, 25), f32),
        ],
    )(sc, tv, base1.reshape(8, 25), noise1.reshape(8, 25),
      base2.reshape(128, 25), noise2.reshape(128, 25))

    w1g = w1n.reshape(8, 5, 5)[:, :, _DXC]
    g1 = jnp.where(_DX_OK[None, None], w1g, 0.0)
    g1 = g1.transpose(1, 2, 0, 3).reshape(5, 32, 256)
    w2g = w2n.reshape(16, 8, 5, 5)[:, :, :, _DXC]
    g2 = jnp.where(_DX_OK[None, None, None], w2g, 0.0)
    g2 = g2.transpose(2, 1, 3, 0, 4).reshape(5, 256, 512)

    fcp = jnp.concatenate([fc_w, jnp.zeros((6, 12544), f32)], axis=0)
    jblk = 256
    nj = 12544 // jblk
    w2mix = pl.pallas_call(
        _wmix_body,
        grid=(nj,),
        in_specs=[
            pl.BlockSpec((16, jblk), lambda j: (0, j)),
            pl.BlockSpec((jblk, 12544), lambda j: (j, 0)),
        ],
        out_specs=pl.BlockSpec((16, 12544), lambda j: (0, 0)),
        out_shape=jax.ShapeDtypeStruct((16, 12544), f32),
        compiler_params=pltpu.CompilerParams(
            dimension_semantics=("arbitrary",),
        ),
    )(fcp, lin_w)

    xpad = jnp.zeros((256, 32, 32), f32)
    xpad = xpad.at[:, 2:30, 0:28].set(x.reshape(256, 28, 28))
    v = pl.pallas_call(
        _conv_body,
        grid=(256 // _BBLK,),
        in_specs=[
            pl.BlockSpec((_BBLK, 32, 32), lambda i: (i, 0, 0)),
            pl.BlockSpec((5, 32, 256), lambda i: (0, 0, 0)),
            pl.BlockSpec((5, 256, 512), lambda i: (0, 0, 0)),
        ],
        out_specs=pl.BlockSpec((_BBLK, 32, 512), lambda i: (i, 0, 0)),
        out_shape=jax.ShapeDtypeStruct((256, 32, 512), f32),
        compiler_params=pltpu.CompilerParams(
            dimension_semantics=("arbitrary",),
        ),
    )(xpad, g1, g2)

    w3 = w2mix.reshape(16, 16, 28, 28)
    w3 = jnp.pad(w3, ((0, 0), (0, 0), (2, 2), (0, 4)))
    wf = w3.transpose(2, 1, 3, 0).reshape(16384, 16)
    fcb = jnp.concatenate([fc_b, jnp.zeros((6,), f32)]).reshape(1, 16)
    y16 = pl.pallas_call(
        _fc_body,
        out_shape=jax.ShapeDtypeStruct((256, 16), f32),
    )(v.reshape(256, 16384), wf, fcb)
    return y16[:, :10]


# speedup vs baseline: 1.5631x; 1.5631x over previous
import math

import jax
import jax.numpy as jnp
import numpy as np
from jax.experimental import pallas as pl
from jax.experimental.pallas import tpu as pltpu

_SC = sum(math.sin(c + 1) for c in range(5))
_SD = sum(1.0 / (math.cos(d + 1e-9) + 1e-9) for d in range(5))
_SE = sum(math.sqrt(e + 1) for e in range(5))
_SMUL = _SC * _SD * _SE

_POOL = 28 * 28 + 1e-9
_BBLK = 16
_NJ = 49

_ci = np.arange(23)
_cj = (_ci * 7919) % 23
_ck = (_cj * 1543) % 23
_M3 = np.zeros((23, 23, 23), np.float32)
_M3[_ci, _cj, _ck] = 1.0

_s = np.arange(256)
_l = np.arange(512)
_AM = (np.arange(8)[None, :] == (_s // 32)[:, None]).astype(np.float32)
_BM = ((_l // 32)[None, :] == np.arange(16)[:, None]).astype(np.float32)
_B1M = ((np.arange(256) // 32)[None, :] == np.arange(8)[:, None]).astype(np.float32)

def _tapmask(nc):
    s = np.arange(nc * 32) % 32
    lx = _l[: 512] % 32
    m = np.zeros((5, nc * 32, 512), np.float32)
    for dx in range(5):
        m[dx] = (
            ((s[:, None] - lx[None, :] + 2) == dx)
            & (s[:, None] < 28) & (lx[None, :] < 28)
        ).astype(np.float32)
    return m

_M5 = _tapmask(8)
_M1 = _tapmask(1)[:, :, :256]


def _mk_w(base, noise, ab, ts):
    acc = base * _SMUL
    acc = acc + ab * jnp.sum(acc, axis=(2, 3), keepdims=True)
    acc = acc + ts
    r = noise
    for _ in range(3):
        r = r * (r + 1e-7)
    k = acc + r
    return k / (jnp.mean(jnp.abs(k)) + 1e-12)


def _rollrows(x, a):
    if a == 0:
        return x
    return jnp.roll(x, -a, axis=0)


def _mega_body(lin_ref, fcw_ref, x_ref, b1_ref, n1_ref, b2_ref, n2_ref,
               a1_ref, b1s_ref, a2_ref, b2s_ref, th1_ref, th2_ref,
               m3_ref, am_ref, bm_ref, b1m_ref, m5_ref, m1_ref,
               w2_ref, v_ref, g1_scr, g2_scr):
    j = pl.program_id(0)

    @pl.when(j == 0)
    def _prep():
        ts1 = jnp.sum(th1_ref[...] * m3_ref[...]) * 1e-12
        ts2 = jnp.sum(th2_ref[...] * m3_ref[...]) * 1e-12
        w1n = _mk_w(b1_ref[...], n1_ref[...], a1_ref[0] * b1s_ref[0], ts1)
        w2n = _mk_w(b2_ref[...], n2_ref[...], a2_ref[0] * b2s_ref[0], ts2)
        for dy in range(5):
            acc1 = jnp.zeros((32, 256), jnp.float32)
            acc2 = jnp.zeros((256, 512), jnp.float32)
            for dx in range(5):
                w1c = w1n[:, 0:1, dy, dx]
                row = jnp.dot(w1c.T, b1m_ref[...],
                              preferred_element_type=jnp.float32)
                acc1 = acc1 + row * m1_ref[dx]
                w2m = w2n[:, :, dy, dx].T
                f = jnp.dot(
                    jnp.dot(am_ref[...], w2m,
                            preferred_element_type=jnp.float32),
                    bm_ref[...], preferred_element_type=jnp.float32,
                )
                acc2 = acc2 + f * m5_ref[dx]
            g1_scr[dy] = acc1
            g2_scr[dy] = acc2
        w2_ref[...] = jnp.zeros_like(w2_ref)

    w2_ref[...] += jnp.dot(
        fcw_ref[...], lin_ref[...], preferred_element_type=jnp.float32
    )

    @pl.when(j < 256 // _BBLK)
    def _conv():
        xb = x_ref[...]
        xt = jnp.transpose(xb, (1, 0, 2))
        xp = jnp.pad(xt, ((2, 2), (0, 0), (0, 4)))
        h1 = jnp.einsum("ybk,kn->ybn", xp, g1_scr[2],
                        preferred_element_type=jnp.float32)
        for ai, a in ((0, -2), (1, -1), (3, 1), (4, 2)):
            h1 = h1 + jnp.einsum("ybk,kn->ybn", _rollrows(xp, a), g1_scr[ai],
                                 preferred_element_type=jnp.float32)
        yid = jax.lax.broadcasted_iota(jnp.int32, (32, 1, 1), 0)
        yok = (yid >= 2) & (yid < 30)
        x1 = jax.lax.broadcasted_iota(jnp.int32, (1, 1, 256), 2) % 32 < 28
        h1 = jnp.where(yok & x1, jnp.maximum(h1, 0.0) / _POOL, 0.0)

        h2 = jnp.einsum("ybk,kn->ybn", h1, g2_scr[2],
                        preferred_element_type=jnp.float32)
        for ai, a in ((0, -2), (1, -1), (3, 1), (4, 2)):
            h2 = h2 + jnp.einsum("ybk,kn->ybn", _rollrows(h1, a), g2_scr[ai],
                                 preferred_element_type=jnp.float32)
        x2 = jax.lax.broadcasted_iota(jnp.int32, (1, 1, 512), 2) % 32 < 28
        h2 = jnp.where(yok & x2, jnp.maximum(h2, 0.0) / _POOL, 0.0)

        ss = jnp.sum(h2 * h2, axis=(0, 2), keepdims=True)
        vn = h2 / (jnp.sqrt(ss) + 1e-20)
        v_ref[...] = vn * (vn + 1e-12)


def _fc_body(v_ref, wf_ref, b_ref, y_ref):
    acc = jnp.dot(v_ref[2], wf_ref[2], preferred_element_type=jnp.float32)
    for yy in range(3, 30):
        acc = acc + jnp.dot(v_ref[yy], wf_ref[yy],
                            preferred_element_type=jnp.float32)
    y = acc + b_ref[...]
    m = jnp.mean(jnp.abs(y))
    y_ref[...] = y / (m + 1e-30)


def kernel(x, base1, a1, b1, thrash1, noise1, base2, a2, b2, thrash2, noise2,
           lin_w, fc_w, fc_b):
    f32 = jnp.float32
    x3 = x.reshape(256, 28, 28)

    smem = pl.BlockSpec(memory_space=pltpu.SMEM)
    full = pl.BlockSpec()
    w2mix, v = pl.pallas_call(
        _mega_body,
        grid=(_NJ,),
        in_specs=[
            pl.BlockSpec((256, 12544), lambda j: (j, 0)),
            pl.BlockSpec((10, 256), lambda j: (0, j)),
            pl.BlockSpec((_BBLK, 28, 28),
                         lambda j: (jnp.minimum(j, 15), 0, 0)),
            full, full, full, full,
            smem, smem, smem, smem,
            full, full,
            full, full, full, full, full, full,
        ],
        out_specs=[
            pl.BlockSpec((10, 12544), lambda j: (0, 0)),
            pl.BlockSpec((32, _BBLK, 512),
                         lambda j: (0, jnp.minimum(j, 15), 0)),
        ],
        out_shape=[
            jax.ShapeDtypeStruct((10, 12544), f32),
            jax.ShapeDtypeStruct((32, 256, 512), f32),
        ],
        scratch_shapes=[
            pltpu.VMEM((5, 32, 256), f32),
            pltpu.VMEM((5, 256, 512), f32),
        ],
        compiler_params=pltpu.CompilerParams(
            dimension_semantics=("arbitrary",),
        ),
    )(lin_w, fc_w, x3, base1, noise1, base2, noise2,
      a1.reshape(1), b1.reshape(1), a2.reshape(1), b2.reshape(1),
      thrash1, thrash2,
      jnp.asarray(_M3), jnp.asarray(_AM), jnp.asarray(_BM),
      jnp.asarray(_B1M), jnp.asarray(_M5), jnp.asarray(_M1))

    w3 = w2mix.reshape(10, 16, 28, 28)
    w3 = jnp.pad(w3, ((0, 0), (0, 0), (2, 2), (0, 4)))
    wf = w3.transpose(2, 1, 3, 0).reshape(32, 512, 10)

    y = pl.pallas_call(
        _fc_body,
        out_shape=jax.ShapeDtypeStruct((256, 10), f32),
    )(v, wf, fc_b.reshape(1, 10))
    return y
